# Initial kernel scaffold; baseline (speedup 1.0000x reference)
import jax
import jax.numpy as jnp
from jax.experimental import pallas as pl


def kernel(x, edge_index1, edge_index2, W_self1, W_neigh1, b1, W_self2, W_neigh2, b2):
    raise NotImplementedError("write your pallas kernel here")


# speedup vs baseline: 4.0821x; 4.0821x over previous
import functools

import jax
import jax.numpy as jnp
from jax import lax
from jax.experimental import pallas as pl
from jax.experimental.pallas import tpu as pltpu
from jax.experimental.pallas import tpu_sc as plsc

N = 10000
D = 128
E = 320000

NC = 2
NS = 16
NW = NC * NS
B = 80
NB = E // B
NP = 10240
RPT = NP // NS
NI = NB // NW
CW = 16


def _agg_body(x_hbm, src_hbm, dst_hbm, zrows_hbm,
              agg_hbm,
              acc, src_v, dst_v, rows_v, sem):
    c = lax.axis_index("c")
    s = lax.axis_index("s")
    w = s * NC + c
    r0 = s * RPT

    pltpu.sync_copy(zrows_hbm.at[pl.ds(r0, RPT)], acc.at[pl.ds(r0, RPT)])
    plsc.subcore_barrier()

    def body(i, carry):
        base = (w + i * NW) * B
        pltpu.sync_copy(src_hbm.at[pl.ds(base, B)], src_v)
        pltpu.sync_copy(dst_hbm.at[pl.ds(base, B)], dst_v)
        pltpu.async_copy(x_hbm.at[src_v], rows_v, sem).wait()
        pltpu.sync_copy(rows_v, acc.at[dst_v], add=True)
        return carry

    lax.fori_loop(0, NI, body, 0)

    plsc.subcore_barrier()
    pltpu.sync_copy(acc.at[pl.ds(r0, RPT)], agg_hbm.at[c, pl.ds(r0, RPT)])


def _cnt_body(dst_hbm, zrows_hbm, cnt_hbm, cacc, dst_v, ones_v):
    c = lax.axis_index("c")
    s = lax.axis_index("s")
    w = s * NC + c
    r0 = s * RPT

    pltpu.sync_copy(zrows_hbm.at[pl.ds(r0, RPT)], cacc.at[pl.ds(r0, RPT)])
    ones16 = jnp.ones((16,), jnp.float32)

    def fill(i, carry):
        for q in range(D // 16):
            ones_v[i, pl.ds(q * 16, 16)] = ones16
        return carry

    lax.fori_loop(0, B, fill, 0)
    plsc.subcore_barrier()

    def body(i, carry):
        base = (w + i * NW) * B
        pltpu.sync_copy(dst_hbm.at[pl.ds(base, B)], dst_v)
        pltpu.sync_copy(ones_v, cacc.at[dst_v], add=True)
        return carry

    lax.fori_loop(0, NI, body, 0)

    plsc.subcore_barrier()
    pltpu.sync_copy(cacc.at[pl.ds(r0, RPT)], cnt_hbm.at[c, pl.ds(r0, RPT)])


def _sc_aggregate(feat, src, dst):
    zrows = jnp.zeros((NP, D), jnp.float32)
    agg, = pl.kernel(
        _agg_body,
        out_type=[
            jax.ShapeDtypeStruct((NC, NP, D), jnp.float32),
        ],
        mesh=plsc.VectorSubcoreMesh(core_axis_name="c", subcore_axis_name="s"),
        scratch_types=[
            pltpu.VMEM_SHARED((NP, D), jnp.float32),
            pltpu.VMEM((B,), jnp.int32),
            pltpu.VMEM((B,), jnp.int32),
            pltpu.VMEM((B, D), jnp.float32),
            pltpu.SemaphoreType.DMA,
        ],
    )(feat, src, dst, zrows)
    cnt, = pl.kernel(
        _cnt_body,
        out_type=[
            jax.ShapeDtypeStruct((NC, NP, D), jnp.float32),
        ],
        mesh=plsc.VectorSubcoreMesh(core_axis_name="c", subcore_axis_name="s"),
        scratch_types=[
            pltpu.VMEM_SHARED((NP, D), jnp.float32),
            pltpu.VMEM((B,), jnp.int32),
            pltpu.VMEM((B, D), jnp.float32),
        ],
    )(dst, zrows)
    return agg, cnt


R = 1000


def _mlp_body(x_ref, agg_ref, cnt_ref, ws_ref, wn_ref, b_ref, out_ref, *, relu):
    cnt = cnt_ref[0][:, :1] + cnt_ref[1][:, :1]
    mean = (agg_ref[0] + agg_ref[1]) / jnp.maximum(cnt, 1.0)
    acc = lax.dot_general(x_ref[...], ws_ref[...], (((1,), (1,)), ((), ())),
                          preferred_element_type=jnp.float32)
    acc = acc + lax.dot_general(mean, wn_ref[...], (((1,), (1,)), ((), ())),
                                preferred_element_type=jnp.float32)
    acc = acc + b_ref[...]
    out_ref[...] = jnp.maximum(acc, 0.0) if relu else acc


def _mlp(x, agg, cnt, W_self, W_neigh, b, relu):
    h = W_self.shape[0]
    return pl.pallas_call(
        functools.partial(_mlp_body, relu=relu),
        grid=(N // R,),
        in_specs=[
            pl.BlockSpec((R, x.shape[1]), lambda i: (i, 0)),
            pl.BlockSpec((NC, R, D), lambda i: (0, i, 0)),
            pl.BlockSpec((NC, R, D), lambda i: (0, i, 0)),
            pl.BlockSpec(W_self.shape, lambda i: (0, 0)),
            pl.BlockSpec(W_neigh.shape, lambda i: (0, 0)),
            pl.BlockSpec((1, h), lambda i: (0, 0)),
        ],
        out_specs=pl.BlockSpec((R, h), lambda i: (i, 0)),
        out_shape=jax.ShapeDtypeStruct((N, h), jnp.float32),
    )(x, agg, cnt, W_self, W_neigh, b.reshape(1, h))


def kernel(x, edge_index1, edge_index2, W_self1, W_neigh1, b1,
           W_self2, W_neigh2, b2):
    src1 = edge_index1[0].reshape(E)
    dst1 = edge_index1[1].reshape(E)
    src2 = edge_index2[0].reshape(E)
    dst2 = edge_index2[1].reshape(E)
    agg1, cnt1 = _sc_aggregate(x, src1, dst1)
    hmid = _mlp(x, agg1, cnt1, W_self1, W_neigh1, b1, relu=True)
    agg2, cnt2 = _sc_aggregate(hmid, src2, dst2)
    return _mlp(hmid, agg2, cnt2, W_self2, W_neigh2, b2, relu=False)


# speedup vs baseline: 4.4718x; 1.0955x over previous
import functools

import jax
import jax.numpy as jnp
from jax import lax
from jax.experimental import pallas as pl
from jax.experimental.pallas import tpu as pltpu
from jax.experimental.pallas import tpu_sc as plsc

N = 10000
D = 128
E = 320000

NC = 2
NS = 16
NW = NC * NS
B = 40
NB = E // B
NP = 10240
RPT = NP // NS
NI = NB // NW
BC = 40
NIC = (E // BC) // NW


def _agg_body(x_hbm, src_hbm, dst_hbm, zrows_hbm,
              agg_hbm,
              acc, s0, s1, d0, d1, r0v, r1v, sem0, sem1):
    c = lax.axis_index("c")
    s = lax.axis_index("s")
    w = s * NC + c
    r0 = s * RPT

    pltpu.sync_copy(zrows_hbm.at[pl.ds(r0, RPT)], acc.at[pl.ds(r0, RPT)])
    plsc.subcore_barrier()

    def load_idx(i, sv, dv):
        base = (w + i * NW) * B
        pltpu.sync_copy(src_hbm.at[pl.ds(base, B)], sv)
        pltpu.sync_copy(dst_hbm.at[pl.ds(base, B)], dv)

    load_idx(0, s0, d0)
    pltpu.async_copy(x_hbm.at[s0], r0v, sem0)

    def body(k, carry):
        load_idx(2 * k + 1, s1, d1)
        pltpu.async_copy(x_hbm.at[s1], r1v, sem1)
        pltpu.make_async_copy(x_hbm.at[s0], r0v, sem0).wait()
        pltpu.sync_copy(r0v, acc.at[d0], add=True)

        @pl.when(k < NI // 2 - 1)
        def _():
            load_idx(2 * k + 2, s0, d0)
            pltpu.async_copy(x_hbm.at[s0], r0v, sem0)

        pltpu.make_async_copy(x_hbm.at[s1], r1v, sem1).wait()
        pltpu.sync_copy(r1v, acc.at[d1], add=True)
        return carry

    lax.fori_loop(0, NI // 2, body, 0)

    plsc.subcore_barrier()
    pltpu.sync_copy(acc.at[pl.ds(r0, RPT)], agg_hbm.at[c, pl.ds(r0, RPT)])


def _cnt_body(dst_hbm, zrows_hbm, cnt_hbm, cacc, d0, d1, ones_v, sem0, sem1):
    c = lax.axis_index("c")
    s = lax.axis_index("s")
    w = s * NC + c
    r0 = s * RPT

    pltpu.sync_copy(zrows_hbm.at[pl.ds(r0, RPT)], cacc.at[pl.ds(r0, RPT)])
    ones16 = jnp.ones((16,), jnp.float32)

    def fill(i, carry):
        for q in range(D // 16):
            ones_v[i, pl.ds(q * 16, 16)] = ones16
        return carry

    lax.fori_loop(0, BC, fill, 0)
    plsc.subcore_barrier()

    def dslice(i):
        return dst_hbm.at[pl.ds((w + i * NW) * BC, BC)]

    pltpu.async_copy(dslice(0), d0, sem0)

    def body(k, carry):
        pltpu.async_copy(dslice(2 * k + 1), d1, sem1)
        pltpu.make_async_copy(dslice(2 * k), d0, sem0).wait()
        pltpu.sync_copy(ones_v, cacc.at[d0], add=True)

        @pl.when(k < NIC // 2 - 1)
        def _():
            pltpu.async_copy(dslice(2 * k + 2), d0, sem0)

        pltpu.make_async_copy(dslice(2 * k + 1), d1, sem1).wait()
        pltpu.sync_copy(ones_v, cacc.at[d1], add=True)
        return carry

    lax.fori_loop(0, NIC // 2, body, 0)

    plsc.subcore_barrier()
    pltpu.sync_copy(cacc.at[pl.ds(r0, RPT)], cnt_hbm.at[c, pl.ds(r0, RPT)])


def _sc_aggregate(feat, src, dst):
    zrows = jnp.zeros((NP, D), jnp.float32)
    agg, = pl.kernel(
        _agg_body,
        out_type=[
            jax.ShapeDtypeStruct((NC, NP, D), jnp.float32),
        ],
        mesh=plsc.VectorSubcoreMesh(core_axis_name="c", subcore_axis_name="s"),
        scratch_types=[
            pltpu.VMEM_SHARED((NP, D), jnp.float32),
            pltpu.VMEM((B,), jnp.int32),
            pltpu.VMEM((B,), jnp.int32),
            pltpu.VMEM((B,), jnp.int32),
            pltpu.VMEM((B,), jnp.int32),
            pltpu.VMEM((B, D), jnp.float32),
            pltpu.VMEM((B, D), jnp.float32),
            pltpu.SemaphoreType.DMA,
            pltpu.SemaphoreType.DMA,
        ],
    )(feat, src, dst, zrows)
    return agg


def _sc_count(dst):
    zrows = jnp.zeros((NP, D), jnp.float32)
    cnt, = pl.kernel(
        _cnt_body,
        out_type=[
            jax.ShapeDtypeStruct((NC, NP, D), jnp.float32),
        ],
        mesh=plsc.VectorSubcoreMesh(core_axis_name="c", subcore_axis_name="s"),
        scratch_types=[
            pltpu.VMEM_SHARED((NP, D), jnp.float32),
            pltpu.VMEM((BC,), jnp.int32),
            pltpu.VMEM((BC,), jnp.int32),
            pltpu.VMEM((BC, D), jnp.float32),
            pltpu.SemaphoreType.DMA,
            pltpu.SemaphoreType.DMA,
        ],
    )(dst, zrows)
    return cnt


R = 1000


def _mlp_body(x_ref, agg_ref, cnt_ref, ws_ref, wn_ref, b_ref, out_ref, *, relu):
    cnt = cnt_ref[0][:, :1] + cnt_ref[1][:, :1]
    mean = (agg_ref[0] + agg_ref[1]) / jnp.maximum(cnt, 1.0)
    acc = lax.dot_general(x_ref[...], ws_ref[...], (((1,), (1,)), ((), ())),
                          preferred_element_type=jnp.float32)
    acc = acc + lax.dot_general(mean, wn_ref[...], (((1,), (1,)), ((), ())),
                                preferred_element_type=jnp.float32)
    acc = acc + b_ref[...]
    out_ref[...] = jnp.maximum(acc, 0.0) if relu else acc


def _mlp(x, agg, cnt, W_self, W_neigh, b, relu):
    h = W_self.shape[0]
    return pl.pallas_call(
        functools.partial(_mlp_body, relu=relu),
        grid=(N // R,),
        in_specs=[
            pl.BlockSpec((R, x.shape[1]), lambda i: (i, 0)),
            pl.BlockSpec((NC, R, D), lambda i: (0, i, 0)),
            pl.BlockSpec((NC, R, D), lambda i: (0, i, 0)),
            pl.BlockSpec(W_self.shape, lambda i: (0, 0)),
            pl.BlockSpec(W_neigh.shape, lambda i: (0, 0)),
            pl.BlockSpec((1, h), lambda i: (0, 0)),
        ],
        out_specs=pl.BlockSpec((R, h), lambda i: (i, 0)),
        out_shape=jax.ShapeDtypeStruct((N, h), jnp.float32),
    )(x, agg, cnt, W_self, W_neigh, b.reshape(1, h))


def kernel(x, edge_index1, edge_index2, W_self1, W_neigh1, b1,
           W_self2, W_neigh2, b2):
    src1 = edge_index1[0].reshape(E)
    dst1 = edge_index1[1].reshape(E)
    src2 = edge_index2[0].reshape(E)
    dst2 = edge_index2[1].reshape(E)
    cnt1 = _sc_count(dst1)
    cnt2 = _sc_count(dst2)
    agg1 = _sc_aggregate(x, src1, dst1)
    hmid = _mlp(x, agg1, cnt1, W_self1, W_neigh1, b1, relu=True)
    agg2 = _sc_aggregate(hmid, src2, dst2)
    return _mlp(hmid, agg2, cnt2, W_self2, W_neigh2, b2, relu=False)
